# baseline (speedup 1.0000x reference)
import functools

import jax
import jax.numpy as jnp
from jax import lax
from jax.experimental import pallas as pl
from jax.experimental.pallas import tpu as pltpu
from jax.experimental.pallas import tpu_sc as plsc

_NC = 2
_NS = 16
_L = 16


def _make_kernel(B, K, b_per_w):
    mesh = plsc.VectorSubcoreMesh(core_axis_name="c", subcore_axis_name="s")

    @functools.partial(
        pl.kernel,
        mesh=mesh,
        compiler_params=pltpu.CompilerParams(
            needs_layout_passes=False, use_tc_tiling_on_sc=False
        ),
        out_type=jax.ShapeDtypeStruct((B,), jnp.float32),
        scratch_types=[
            pltpu.VMEM((b_per_w,), jnp.int32),
            pltpu.VMEM((b_per_w,), jnp.int32),
            pltpu.VMEM((b_per_w, K), jnp.float32),
            pltpu.VMEM((b_per_w, K), jnp.float32),
            pltpu.VMEM((b_per_w * _L,), jnp.float32),
            pltpu.VMEM((b_per_w,), jnp.float32),
            pltpu.SemaphoreType.DMA,
            pltpu.SemaphoreType.DMA,
        ],
    )
    def mf(vids_hbm, mids_hbm, vtab_hbm, mtab_hbm, out_hbm,
           vidx, midx, urows, vrows, partials, outv, sem_u, sem_v):
        wid = lax.axis_index("s") * _NC + lax.axis_index("c")
        base = wid * b_per_w
        pltpu.sync_copy(vids_hbm.at[pl.ds(base, b_per_w)], vidx)
        pltpu.sync_copy(mids_hbm.at[pl.ds(base, b_per_w)], midx)
        cu = pltpu.async_copy(vtab_hbm.at[vidx], urows, sem_u)
        cv = pltpu.async_copy(mtab_hbm.at[midx], vrows, sem_v)
        cu.wait()
        cv.wait()

        lanes = lax.iota(jnp.int32, _L)

        def row_body(b, _):
            p = jnp.zeros((_L,), jnp.float32)
            for k0 in range(0, K, _L):
                u = urows[b, pl.ds(k0, _L)]
                v = vrows[b, pl.ds(k0, _L)]
                p = p + u * v
            partials[pl.ds(b * _L, _L)] = p
            return 0

        lax.fori_loop(0, b_per_w, row_body, 0)

        def red_body(g, _):
            bidx = g * (_L * _L) + lanes * _L
            acc = plsc.load_gather(partials, [bidx])
            for l in range(1, _L):
                acc = acc + plsc.load_gather(partials, [bidx + l])
            outv[pl.ds(g * _L, _L)] = acc
            return 0

        lax.fori_loop(0, b_per_w // _L, red_body, 0)
        pltpu.sync_copy(outv, out_hbm.at[pl.ds(base, b_per_w)])

    return mf


def kernel(viewer_ids, movie_ids, viewer_table, movie_table):
    B = viewer_ids.shape[0]
    K = viewer_table.shape[1]
    b_per_w = B // (_NC * _NS)
    mf = _make_kernel(B, K, b_per_w)
    return mf(viewer_ids, movie_ids, viewer_table, movie_table)
